# baseline (speedup 1.0000x reference)
import functools

import jax
import jax.numpy as jnp
import numpy as np
from jax import lax
from jax.experimental import pallas as pl
from jax.experimental.pallas import tpu as pltpu
from jax.experimental.pallas import tpu_sc as plsc

_B = 16384
_C = 1000
_K = 26

_INFO = plsc.get_sparse_core_info()
_NC = _INFO.num_cores
_NS = _INFO.num_subcores
_L = _INFO.num_lanes
_NW = _NC * _NS
_ROWS = _B // _NW
_CHUNK = 32
_NCHUNK = _ROWS // _CHUNK
_GROUPS = _CHUNK * _K // _L
_UN = 4
_CLEAR_UN = 8

_ROW_TAB = (np.arange(_CHUNK * _K) // _K).astype(np.int32)

_mesh = plsc.VectorSubcoreMesh(core_axis_name="c", subcore_axis_name="s")


@functools.partial(
    pl.kernel,
    mesh=_mesh,
    compiler_params=pltpu.CompilerParams(
        needs_layout_passes=False, use_tc_tiling_on_sc=True
    ),
    out_type=jax.ShapeDtypeStruct((_B, _C), jnp.float32),
    scratch_types=[
        pltpu.VMEM((_ROWS * _K,), jnp.int32),
        pltpu.VMEM((_CHUNK, _C), jnp.float32),
        pltpu.VMEM((_CHUNK, _C), jnp.float32),
        pltpu.VMEM((_CHUNK * _K,), jnp.int32),
        pltpu.SemaphoreType.DMA,
        pltpu.SemaphoreType.DMA,
    ],
)
def _multihot(idx_hbm, rowtab_hbm, out_hbm, idx_v, buf0, buf1, tab_v, sem0, sem1):
    wid = lax.axis_index("s") * _NC + lax.axis_index("c")
    rbase = wid * _ROWS

    pltpu.sync_copy(idx_hbm.at[pl.ds(rbase * _K, _ROWS * _K)], idx_v)
    pltpu.sync_copy(rowtab_hbm, tab_v)

    zeros = jnp.zeros((_L,), jnp.float32)
    ones = jnp.full((_L,), 1.0, jnp.float32)

    def _clear(buf):
        def row_body(r, carry):
            def col_body(j, carry2):
                for u in range(_CLEAR_UN):
                    buf[r, pl.ds((j * _CLEAR_UN + u) * _L, _L)] = zeros
                return carry2

            lax.fori_loop(0, (_C // _L) // _CLEAR_UN, col_body, 0)
            for u in range(_C // _L - (_C // _L) // _CLEAR_UN * _CLEAR_UN):
                buf[r, pl.ds(((_C // _L) // _CLEAR_UN * _CLEAR_UN + u) * _L, _L)] = zeros
            buf[r, pl.ds(_C - _L, _L)] = zeros
            return carry

        lax.fori_loop(0, _CHUNK, row_body, 0)

    def _scatter(buf, c, val):
        base = c * (_CHUNK * _K)

        def body(i, carry):
            p = i * (_UN * _L)
            for u in range(_UN):
                col = idx_v[pl.ds(base + p + u * _L, _L)]
                row = tab_v[pl.ds(p + u * _L, _L)]
                plsc.store_scatter(buf, [row, col], val)
            return carry

        lax.fori_loop(0, _GROUPS // _UN, body, 0)

    def _stream(buf, c, sem):
        return pltpu.async_copy(
            buf, out_hbm.at[pl.ds(rbase + c * _CHUNK, _CHUNK), :], sem
        )

    bufs = (buf0, buf1)
    sems = (sem0, sem1)
    copies = [None] * _NCHUNK
    _clear(buf0)
    _clear(buf1)
    for c in range(_NCHUNK):
        b = c % 2
        if c >= 2:
            copies[c - 2].wait()
            _scatter(bufs[b], c - 2, zeros)
        _scatter(bufs[b], c, ones)
        copies[c] = _stream(bufs[b], c, sems[b])
    copies[_NCHUNK - 2].wait()
    copies[_NCHUNK - 1].wait()


def kernel(inputs):
    flat = inputs.reshape(_B * _K)
    return _multihot(flat, jnp.asarray(_ROW_TAB))


# speedup vs baseline: 1.1459x; 1.1459x over previous
import functools

import jax
import jax.numpy as jnp
from jax import lax
from jax.experimental import pallas as pl
from jax.experimental.pallas import tpu as pltpu
from jax.experimental.pallas import tpu_sc as plsc

_B = 16384
_C = 1000
_K = 26

_INFO = plsc.get_sparse_core_info()
_NC = _INFO.num_cores
_NS = _INFO.num_subcores
_L = _INFO.num_lanes
_NW = _NC * _NS
_ROWS = _B // _NW
_CHUNK = 32
_NCHUNK = _ROWS // _CHUNK
_HALVES = _CHUNK // _L

_mesh = plsc.VectorSubcoreMesh(core_axis_name="c", subcore_axis_name="s")


@functools.partial(
    pl.kernel,
    mesh=_mesh,
    compiler_params=pltpu.CompilerParams(needs_layout_passes=False),
    out_type=jax.ShapeDtypeStruct((_B, _C), jnp.float32),
    scratch_types=[
        pltpu.VMEM((_K, _ROWS), jnp.int32),
        pltpu.VMEM((_CHUNK, _C), jnp.float32),
        pltpu.VMEM((_CHUNK, _C), jnp.float32),
        pltpu.SemaphoreType.DMA,
        pltpu.SemaphoreType.DMA,
    ],
)
def _multihot(idx_hbm, out_hbm, idx_v, buf0, buf1, sem0, sem1):
    wid = lax.axis_index("s") * _NC + lax.axis_index("c")
    rbase = wid * _ROWS

    pltpu.sync_copy(idx_hbm.at[pl.ds(0, _K), pl.ds(rbase, _ROWS)], idx_v)

    lanes = lax.iota(jnp.int32, _L)
    zeros = jnp.zeros((_L,), jnp.float32)
    ones = jnp.full((_L,), 1.0, jnp.float32)

    def _clear(buf):
        def body(r, carry):
            def col_body(j, carry2):
                for u in range(8):
                    buf[r, pl.ds((j * 8 + u) * _L, _L)] = zeros
                return carry2

            lax.fori_loop(0, 7, col_body, 0)
            for u in range(6):
                buf[r, pl.ds((56 + u) * _L, _L)] = zeros
            buf[r, pl.ds(_C - _L, _L)] = zeros
            return carry

        lax.fori_loop(0, _CHUNK, body, 0)

    def _scatter(buf, c, val):
        def body(k, carry):
            for h in range(_HALVES):
                v = idx_v[k, pl.ds(c * _CHUNK + h * _L, _L)]
                plsc.store_scatter(buf, [lanes + (h * _L), v], val)
            return carry

        lax.fori_loop(0, _K, body, 0)

    def _stream(buf, c, sem):
        return pltpu.async_copy(
            buf, out_hbm.at[pl.ds(rbase + c * _CHUNK, _CHUNK), :], sem
        )

    bufs = (buf0, buf1)
    sems = (sem0, sem1)
    copies = [None] * _NCHUNK
    _clear(buf0)
    _clear(buf1)
    for c in range(_NCHUNK):
        b = c % 2
        if c >= 2:
            copies[c - 2].wait()
            _scatter(bufs[b], c - 2, zeros)
        _scatter(bufs[b], c, ones)
        copies[c] = _stream(bufs[b], c, sems[b])
    copies[_NCHUNK - 2].wait()
    copies[_NCHUNK - 1].wait()


def kernel(inputs):
    return _multihot(inputs.T)


# speedup vs baseline: 1.2332x; 1.0762x over previous
import functools

import jax
import jax.numpy as jnp
from jax import lax
from jax.experimental import pallas as pl
from jax.experimental.pallas import tpu as pltpu
from jax.experimental.pallas import tpu_sc as plsc

_B = 16384
_C = 1000
_K = 26

_INFO = plsc.get_sparse_core_info()
_NC = _INFO.num_cores
_NS = _INFO.num_subcores
_L = _INFO.num_lanes
_SHARD = _B // _NS
_CB = 128
_NCB = _SHARD // _CB
_PR = 8
_H = 504
_HOFF = _C - _H

_mesh = plsc.VectorSubcoreMesh(core_axis_name="c", subcore_axis_name="s")


@functools.partial(
    pl.kernel,
    mesh=_mesh,
    compiler_params=pltpu.CompilerParams(needs_layout_passes=False),
    out_type=jax.ShapeDtypeStruct((_C, _B), jnp.float32),
    scratch_types=[
        pltpu.VMEM((_PR, _CB), jnp.int32),
        pltpu.VMEM((_H, _CB), jnp.float32),
        pltpu.VMEM((_H, _CB), jnp.float32),
        pltpu.SemaphoreType.DMA,
        pltpu.SemaphoreType.DMA,
    ],
)
def _multihot(idx_hbm, out_hbm, idx_v, buf0, buf1, sem0, sem1):
    shard = lax.axis_index("s")
    half = lax.axis_index("c")
    colbase = shard * _SHARD

    lanes = lax.iota(jnp.int32, _L)
    zeros = jnp.zeros((_L,), jnp.float32)
    ones = jnp.full((_L,), 1.0, jnp.float32)
    clo = half * _HOFF
    clo_v = jnp.zeros((_L,), jnp.int32) + clo

    def _clear(buf):
        def body(r, carry):
            for u in range(_CB // _L):
                buf[r, pl.ds(u * _L, _L)] = zeros
            return carry

        lax.fori_loop(0, _H, body, 0)

    def _scatter(buf, cb, val):
        for p0 in range(0, _K, _PR):
            rows = min(_PR, _K - p0)
            pltpu.sync_copy(
                idx_hbm.at[pl.ds(p0, rows), pl.ds(colbase + cb * _CB, _CB)],
                idx_v.at[pl.ds(0, rows), pl.ds(0, _CB)],
            )

            def body(k, carry):
                for g in range(_CB // _L):
                    v = idx_v[k, pl.ds(g * _L, _L)]
                    c0 = g * _L + lanes
                    m = (v >= clo_v) & (v < clo_v + _H)
                    plsc.store_scatter(buf, [v - clo_v, c0], val, mask=m)
                return carry

            lax.fori_loop(0, rows, body, 0)

    bufs = (buf0, buf1)
    sems = (sem0, sem1)
    copies = [None] * _NCB
    _clear(buf0)
    _clear(buf1)
    for cb in range(_NCB):
        b = cb % 2
        if cb >= 2:
            copies[cb - 2].wait()
            _scatter(bufs[b], cb - 2, zeros)
        _scatter(bufs[b], cb, ones)
        copies[cb] = pltpu.async_copy(
            bufs[b],
            out_hbm.at[pl.ds(clo, _H), pl.ds(colbase + cb * _CB, _CB)],
            sems[b],
        )
    copies[_NCB - 2].wait()
    copies[_NCB - 1].wait()


def kernel(inputs):
    out_t = _multihot(inputs.T)
    return out_t.T
